# Initial kernel scaffold; baseline (speedup 1.0000x reference)
import jax
import jax.numpy as jnp
from jax.experimental import pallas as pl


def kernel(x, table, gw1, gb1, gw2, gb2, gw3, gb3, cw, cb):
    raise NotImplementedError("write your pallas kernel here")


# speedup vs baseline: 9.2380x; 9.2380x over previous
import functools

import jax
import jax.numpy as jnp
import numpy as np
from jax import lax
from jax.experimental import pallas as pl
from jax.experimental.pallas import tpu as pltpu
from jax.experimental.pallas import tpu_sc as plsc

_L = 10
_T = 65536
_F = 4
_N = 524288
_HID = 16
_DIN = _L * _F
_P1 = np.int32(np.int64(2654435761) - (1 << 32))
_P2 = np.int32(805459861)
_RES = [int(np.floor(14 * (1.5 ** l))) for l in range(_L)]

_NC = 2
_NS = 16
_NW = _NC * _NS
_PW = _N // _NW
_C = 512
_NCH = _PW // _C
_SEG = 8192
_SEGSH = 13

def _num_segments(l):
    R = _RES[l]
    S = R + 1
    rows_used = S ** 3 if S ** 3 <= _T else _T
    return -(-rows_used // _SEG)

_V = _L * _T
_E = 8192

_MESH = dict(core_axis_name="c", subcore_axis_name="s",
             num_cores=_NC, num_subcores=_NS)
_SC_PARAMS = pltpu.CompilerParams(
    needs_layout_passes=False, use_tc_tiling_on_sc=False)


def _sc_pack_table(tbl1d):
    mesh = plsc.VectorSubcoreMesh(**_MESH)
    share = _V * _F // _NW
    nch = share // _E

    @functools.partial(
        pl.kernel,
        out_type=jax.ShapeDtypeStruct((_V, _F), jnp.float32),
        mesh=mesh,
        compiler_params=_SC_PARAMS,
        scratch_types=[
            pltpu.VMEM((_E,), jnp.float32),
            pltpu.VMEM((_E // _F, _F), jnp.float32),
        ],
    )
    def pack(t_hbm, out_hbm, abuf, bbuf):
        wid = lax.axis_index("s") * _NC + lax.axis_index("c")
        base = wid * share
        ii = lax.iota(jnp.int32, 16)

        def chunk_body(ch, carry):
            off = base + ch * _E
            pltpu.sync_copy(t_hbm.at[pl.ds(off, _E)], abuf)

            def sc_body(i, _):
                g = ii + i * 16
                v = abuf[pl.ds(i * 16, 16)]
                plsc.store_scatter(bbuf, [g >> 2, g & 3], v)
                return 0

            lax.fori_loop(0, _E // 16, sc_body, 0)
            pltpu.sync_copy(bbuf, out_hbm.at[pl.ds(off // _F, _E // _F), :])
            return carry

        lax.fori_loop(0, nch, chunk_body, 0)

    return pack(tbl1d)


def _sc_encode(xcat, tbl):
    mesh = plsc.VectorSubcoreMesh(**_MESH)

    @functools.partial(
        pl.kernel,
        out_type=jax.ShapeDtypeStruct((_DIN * _N,), jnp.float32),
        mesh=mesh,
        compiler_params=_SC_PARAMS,
        scratch_types=[
            pltpu.VMEM((3, _C), jnp.float32),
            pltpu.VMEM((8 * _C,), jnp.int32),
            pltpu.VMEM((8 * _C,), jnp.float32),
            pltpu.VMEM((_SEG, _F), jnp.float32),
            pltpu.VMEM((_DIN, _C), jnp.float32),
            pltpu.SemaphoreType.DMA,
        ],
    )
    def encode(x_hbm, tbl_hbm, pe_hbm, xbuf, idxbuf, wbuf, tbuf, pebuf,
               osem):
        wid = lax.axis_index("s") * _NC + lax.axis_index("c")
        base = wid * _PW
        ii = lax.iota(jnp.int32, 16)

        def chunk_body(ch, carry):
            cbase = base + ch * _C
            for d in range(3):
                pltpu.sync_copy(x_hbm.at[pl.ds(d * _N + cbase, _C)],
                                xbuf.at[d])

            for l in range(_L):
                R = _RES[l]
                S = R + 1
                dense = S ** 3 <= _T

                def idx_body(i, _, R=R, S=S, dense=dense):
                    s = i * 16
                    xv = xbuf[0, pl.ds(s, 16)]
                    yv = xbuf[1, pl.ds(s, 16)]
                    zv = xbuf[2, pl.ds(s, 16)]
                    fR = jnp.float32(R)
                    px = xv * fR
                    py = yv * fR
                    pz = zv * fR
                    ix0 = px.astype(jnp.int32)
                    iy0 = py.astype(jnp.int32)
                    iz0 = pz.astype(jnp.int32)
                    wx = px - ix0.astype(jnp.float32)
                    wy = py - iy0.astype(jnp.float32)
                    wz = pz - iz0.astype(jnp.float32)
                    ix1 = jnp.minimum(ix0 + 1, R)
                    iy1 = jnp.minimum(iy0 + 1, R)
                    iz1 = jnp.minimum(iz0 + 1, R)
                    if dense:
                        ax = (ix0, ix1)
                        by = (iy0 * S, iy1 * S)
                        cz = (iz0 * (S * S), iz1 * (S * S))
                    else:
                        ax = (ix0, ix1)
                        by = (iy0 * _P1, iy1 * _P1)
                        cz = (iz0 * _P2, iz1 * _P2)
                    wxs = (jnp.float32(1.0) - wx, wx)
                    wyz = {}
                    for byi in range(2):
                        wyv = wy if byi else (jnp.float32(1.0) - wy)
                        for bzi in range(2):
                            wzv = wz if bzi else (jnp.float32(1.0) - wz)
                            wyz[(byi, bzi)] = wyv * wzv
                    for c in range(8):
                        bx, byi, bzi = c & 1, (c >> 1) & 1, (c >> 2) & 1
                        if dense:
                            idx = ax[bx] + by[byi] + cz[bzi]
                        else:
                            idx = (ax[bx] ^ by[byi] ^ cz[bzi]) & jnp.int32(65535)
                        wc = wxs[bx] * wyz[(byi, bzi)]
                        flat = c * _C + s
                        idxbuf[pl.ds(flat, 16)] = idx
                        wbuf[pl.ds(flat, 16)] = wc
                    return 0

                lax.fori_loop(0, _C // 16, idx_body, 0)

                nseg = _num_segments(l)
                if nseg == 1:
                    pltpu.sync_copy(
                        tbl_hbm.at[pl.ds(l * _T, _SEG), :], tbuf)

                    def acc1_body(i, _, l=l):
                        s = i * 16
                        rls = [idxbuf[pl.ds(c * _C + s, 16)]
                               for c in range(8)]
                        wms = [wbuf[pl.ds(c * _C + s, 16)]
                               for c in range(8)]
                        for f in range(_F):
                            fv = jnp.full((16,), f, jnp.int32)
                            acc = None
                            for c in range(8):
                                v = plsc.load_gather(tbuf, [rls[c], fv])
                                acc = (v * wms[c] if acc is None
                                       else acc + v * wms[c])
                            pebuf[l * _F + f, pl.ds(s, 16)] = acc
                        return 0

                    lax.fori_loop(0, _C // 16, acc1_body, 0)
                else:
                    def zero_body(i, _, l=l):
                        z = jnp.zeros((16,), jnp.float32)
                        for f in range(_F):
                            pebuf[l * _F + f, pl.ds(i * 16, 16)] = z
                        return 0

                    lax.fori_loop(0, _C // 16, zero_body, 0)

                    def seg_body(si, _, l=l):
                        pltpu.sync_copy(
                            tbl_hbm.at[pl.ds(l * _T + si * _SEG, _SEG), :],
                            tbuf)

                        def acc_body(i, _):
                            s = i * 16
                            rls, wms = [], []
                            for c in range(8):
                                rv = idxbuf[pl.ds(c * _C + s, 16)]
                                wv = wbuf[pl.ds(c * _C + s, 16)]
                                m = (rv >> _SEGSH) == si
                                rls.append(rv & jnp.int32(_SEG - 1))
                                wms.append(
                                    jnp.where(m, wv, jnp.float32(0.0)))
                            for f in range(_F):
                                fv = jnp.full((16,), f, jnp.int32)
                                acc = None
                                for c in range(8):
                                    v = plsc.load_gather(tbuf, [rls[c], fv])
                                    acc = (v * wms[c] if acc is None
                                           else acc + v * wms[c])
                                acc = acc + pebuf[l * _F + f, pl.ds(s, 16)]
                                pebuf[l * _F + f, pl.ds(s, 16)] = acc
                            return 0

                        lax.fori_loop(0, _C // 16, acc_body, 0)
                        return 0

                    lax.fori_loop(0, nseg, seg_body, 0)

            def out_fire(k, _):
                pltpu.async_copy(
                    pebuf.at[k], pe_hbm.at[pl.ds(k * _N + cbase, _C)], osem)
                return 0

            lax.fori_loop(0, _DIN, out_fire, 0)

            def out_drain(k, _):
                pltpu.make_async_copy(
                    pebuf.at[0], pe_hbm.at[pl.ds(cbase, _C)], osem).wait()
                return 0

            lax.fori_loop(0, _DIN, out_drain, 0)
            return carry

        lax.fori_loop(0, _NCH, chunk_body, 0)

    return encode(xcat, tbl)


def _softplus10(h):
    y = 10.0 * h
    return (jnp.maximum(y, 0.0) + jnp.log(1.0 + jnp.exp(-jnp.abs(y)))) * 0.1


def _mlp_body(pe_ref, w1, b1, w2, b2, w3, b3, cwr, cbr, sdf_ref, col_ref):
    pe = pe_ref[...]
    dn = (((0,), (0,)), ((), ()))
    h = _softplus10(lax.dot_general(w1[...], pe, dn,
                                    preferred_element_type=jnp.float32) + b1[...])
    h = _softplus10(lax.dot_general(w2[...], h, dn,
                                    preferred_element_type=jnp.float32) + b2[...])
    sdf_ref[...] = lax.dot_general(w3[...], h, dn,
                                   preferred_element_type=jnp.float32) + b3[...]
    col_ref[...] = lax.dot_general(cwr[...], pe, dn,
                                   preferred_element_type=jnp.float32) + cbr[...]


def _tc_mlp(pe, gw1, gb1, gw2, gb2, gw3, gb3, cw, cb):
    bn = 4096
    grid = (_N // bn,)
    full = lambda a: pl.BlockSpec(a.shape, lambda i: (0, 0))
    return pl.pallas_call(
        _mlp_body,
        grid=grid,
        in_specs=[
            pl.BlockSpec((_DIN, bn), lambda i: (0, i)),
            full(gw1), full(gb1), full(gw2), full(gb2),
            full(gw3), full(gb3), full(cw), full(cb),
        ],
        out_specs=[
            pl.BlockSpec((1, bn), lambda i: (0, i)),
            pl.BlockSpec((3, bn), lambda i: (0, i)),
        ],
        out_shape=[
            jax.ShapeDtypeStruct((1, _N), jnp.float32),
            jax.ShapeDtypeStruct((3, _N), jnp.float32),
        ],
    )(pe, gw1, gb1, gw2, gb2, gw3, gb3, cw, cb)


def kernel(x, table, gw1, gb1, gw2, gb2, gw3, gb3, cw, cb):
    xcat = x.T.reshape(-1)
    tbl = _sc_pack_table(table.reshape(-1))
    pe1d = _sc_encode(xcat, tbl)
    pe = pe1d.reshape(_DIN, _N)
    sdf2, col2 = _tc_mlp(
        pe, gw1, gb1.reshape(-1, 1), gw2, gb2.reshape(-1, 1),
        gw3, gb3.reshape(-1, 1), cw, cb.reshape(-1, 1))
    return sdf2.reshape(-1), col2.T
